# baseline (speedup 1.0000x reference)
import functools

import jax
import jax.numpy as jnp
from jax import lax
from jax.experimental import pallas as pl
from jax.experimental.pallas import tpu as pltpu
from jax.experimental.pallas import tpu_sc as plsc

_VOCAB = 1000000
_D = 32
_HD = 16
_H = 64
_B = 16384
_L = 200

_NC = 2
_NS = 16
_B_PER_T = _B // _NS
_V_PER_T = _VOCAB // _NS
_CHUNK = 128
_UNROLL = 25

_TCOLS = 1024
_G_PER_T = -(-(_VOCAB // 16) // _NS)
_NCHUNK = -(-_G_PER_T // (_TCOLS // 16))


def _tr_body(embt_hbm, t0_hbm, t1_hbm, in0_v, in1_v, ob0_v, ob1_v,
             si0, si1, so0, so1):
    cid = lax.axis_index("c")
    sid = lax.axis_index("s")
    lanes = lax.iota(jnp.int32, 16)

    def transpose_half(d_off, t_hbm):
        inb = (in0_v, in1_v)
        outb = (ob0_v, ob1_v)
        sin = (si0, si1)
        sout = (so0, so1)
        base_col = sid * _G_PER_T * 16

        def col0_of(c):
            return jnp.minimum(base_col + _TCOLS * c, _VOCAB - _TCOLS)

        def in_copy(c, b):
            return pltpu.make_async_copy(
                embt_hbm.at[pl.ds(d_off, _HD), pl.ds(col0_of(c), _TCOLS)],
                inb[b], sin[b])

        def out_copy(c, b):
            return pltpu.make_async_copy(
                outb[b], t_hbm.at[pl.ds(col0_of(c), _TCOLS), :], sout[b])

        for b in range(2):
            in_copy(b, b).start()

        def chunk_body(i, carry):
            for b in range(2):
                c = 2 * i + b
                in_copy(c, b).wait()

                @pl.when(c >= 2)
                def _():
                    out_copy(c - 2, b).wait()

                def grp_body(j, carry2):
                    rowsel = lanes + 16 * j
                    for d in range(_HD):
                        v = inb[b][d, pl.ds(16 * j, 16)]
                        plsc.store_scatter(
                            outb[b], [rowsel, jnp.full((16,), d, jnp.int32)],
                            v)
                    return carry2

                lax.fori_loop(0, _TCOLS // 16, grp_body, 0)
                out_copy(c, b).start()

                @pl.when(c + 2 < _NCHUNK)
                def _():
                    in_copy(c + 2, b).start()
            return carry

        lax.fori_loop(0, _NCHUNK // 2, chunk_body, 0)
        for b in range(2):
            out_copy(_NCHUNK - 2 + b, b).wait()

    @pl.when(cid == 0)
    def _():
        transpose_half(0, t0_hbm)

    @pl.when(cid == 1)
    def _():
        transpose_half(_HD, t1_hbm)


_transpose = functools.partial(
    pl.kernel,
    mesh=plsc.VectorSubcoreMesh(core_axis_name="c", subcore_axis_name="s"),
    out_type=[
        jax.ShapeDtypeStruct((_VOCAB, _HD), jnp.float32),
        jax.ShapeDtypeStruct((_VOCAB, _HD), jnp.float32),
    ],
    scratch_types=[
        pltpu.VMEM((_HD, _TCOLS), jnp.float32),
        pltpu.VMEM((_HD, _TCOLS), jnp.float32),
        pltpu.VMEM((_TCOLS, _HD), jnp.float32),
        pltpu.VMEM((_TCOLS, _HD), jnp.float32),
        pltpu.SemaphoreType.DMA,
        pltpu.SemaphoreType.DMA,
        pltpu.SemaphoreType.DMA,
        pltpu.SemaphoreType.DMA,
    ],
    compiler_params=pltpu.CompilerParams(use_tc_tiling_on_sc=False,
                                         needs_layout_passes=False),
)(_tr_body)


def _pool_body(x_hbm, g0_hbm, g1_hbm, p0_hbm, p1_hbm,
               idx_v, rows0_v, rows1_v, stage_v, sem0, sem1):
    cid = lax.axis_index("c")
    sid = lax.axis_index("s")

    def pool(t_hbm, p_hbm):
        rows = (rows0_v, rows1_v)
        sems = (sem0, sem1)
        base = sid * _B_PER_T

        def gather(r, b):
            c1 = pltpu.make_async_copy(
                t_hbm.at[idx_v.at[r, pl.ds(0, 128)]],
                rows[b].at[pl.ds(0, 128), :], sems[b])
            c2 = pltpu.make_async_copy(
                t_hbm.at[idx_v.at[r, pl.ds(128, _L - 128)]],
                rows[b].at[pl.ds(128, _L - 128), :], sems[b])
            return c1, c2

        def reduce_store(r, b):
            def red_body(j, accs):
                a0, a1 = accs
                for k in range(_UNROLL):
                    row = j * _UNROLL + k
                    if k % 2 == 0:
                        a0 = a0 + rows[b][row, :]
                    else:
                        a1 = a1 + rows[b][row, :]
                return a0, a1

            z = jnp.zeros((_HD,), jnp.float32)
            a0, a1 = lax.fori_loop(0, _L // _UNROLL, red_body, (z, z))
            stage_v[r, :] = a0 + a1

        for c in range(_B_PER_T // _CHUNK):
            cbase = base + c * _CHUNK
            pltpu.sync_copy(x_hbm.at[pl.ds(cbase, _CHUNK), :], idx_v)
            for b in range(2):
                c1, c2 = gather(b, b)
                c1.start()
                c2.start()

            def pair_body(i, carry):
                for b in range(2):
                    r = 2 * i + b
                    c1, c2 = gather(r, b)
                    c1.wait()
                    c2.wait()
                    reduce_store(r, b)

                    @pl.when(r + 2 < _CHUNK)
                    def _():
                        n1, n2 = gather(r + 2, b)
                        n1.start()
                        n2.start()
                return carry

            lax.fori_loop(0, _CHUNK // 2, pair_body, 0)
            pltpu.sync_copy(stage_v, p_hbm.at[pl.ds(cbase, _CHUNK), :])

    @pl.when(cid == 0)
    def _():
        pool(g0_hbm, p0_hbm)

    @pl.when(cid == 1)
    def _():
        pool(g1_hbm, p1_hbm)


_pool = functools.partial(
    pl.kernel,
    mesh=plsc.VectorSubcoreMesh(core_axis_name="c", subcore_axis_name="s"),
    out_type=[
        jax.ShapeDtypeStruct((_B, _HD), jnp.float32),
        jax.ShapeDtypeStruct((_B, _HD), jnp.float32),
    ],
    scratch_types=[
        pltpu.VMEM((_CHUNK, _L), jnp.int32),
        pltpu.VMEM((_L, _HD), jnp.float32),
        pltpu.VMEM((_L, _HD), jnp.float32),
        pltpu.VMEM((_CHUNK, _HD), jnp.float32),
        pltpu.SemaphoreType.DMA,
        pltpu.SemaphoreType.DMA,
    ],
    compiler_params=pltpu.CompilerParams(use_tc_tiling_on_sc=False),
)(_pool_body)


def _mlp_body(p0_ref, p1_ref, w1a_ref, w1b_ref, b1_ref, w2_ref, b2_ref,
              o_ref):
    s0 = p0_ref[...] * (1.0 / _L)
    s1 = p1_ref[...] * (1.0 / _L)
    h = (jnp.dot(s0, w1a_ref[...], preferred_element_type=jnp.float32) +
         jnp.dot(s1, w1b_ref[...], preferred_element_type=jnp.float32))
    h = jnp.maximum(h + b1_ref[...], 0.0)
    o = jnp.dot(h, w2_ref[...], preferred_element_type=jnp.float32)
    o_ref[...] = jax.nn.sigmoid(o + b2_ref[...])


def _mlp(p0, p1, w1, b1, w2, b2):
    return pl.pallas_call(
        _mlp_body,
        out_shape=jax.ShapeDtypeStruct((_B, 1), jnp.float32),
    )(p0, p1, w1[:_HD], w1[_HD:], b1.reshape(1, _H), w2, b2.reshape(1, 1))


def kernel(x, emb, W1, b1, W2, b2):
    t0, t1 = _transpose(emb.T)
    p0, p1 = _pool(x, t0, t1)
    return _mlp(p0, p1, W1, b1, W2, b2)


# speedup vs baseline: 3.3922x; 3.3922x over previous
import functools

import jax
import jax.numpy as jnp
from jax import lax
from jax.experimental import pallas as pl
from jax.experimental.pallas import tpu as pltpu
from jax.experimental.pallas import tpu_sc as plsc

_VOCAB = 1000000
_D = 32
_H = 64
_B = 16384
_L = 200

_NC = 2
_NS = 16
_NW = _NC * _NS
_B_PER_W = _B // _NW
_CHUNK = 256
_UNROLL = 25

_W = 512
_SPLIT = 499968
_EDGE = (_VOCAB // 128) * 128
_NWIN = 62


def _tr_body(embt_hbm, tail_hbm, t_hbm, in0_v, in1_v, ob0_v, ob1_v, tail_v,
             si0, si1, so0, so1):
    cid = lax.axis_index("c")
    sid = lax.axis_index("s")
    lanes = lax.iota(jnp.int32, 16)
    inb = (in0_v, in1_v)
    outb = (ob0_v, ob1_v)
    sin = (si0, si1)
    sout = (so0, so1)

    lo = jnp.where(cid == 0, 0, _SPLIT)
    hi = jnp.where(cid == 0, _SPLIT, _EDGE)
    span = hi - lo
    cbase = lo + ((sid * span) // (_NS * 128)) * 128
    top = hi - _W

    def in_copy(w, b):
        col0 = jnp.minimum(cbase + _W * w, top)
        return col0, pltpu.make_async_copy(
            embt_hbm.at[:, pl.ds(col0, _W)], inb[b], sin[b])

    def out_copy(w, b):
        col0 = jnp.minimum(cbase + _W * w, top)
        return pltpu.make_async_copy(
            outb[b], t_hbm.at[pl.ds(col0 * _D, _W * _D)], sout[b])

    for b in range(2):
        in_copy(b, b)[1].start()

    def win_body(i, carry):
        for b in range(2):
            w = 2 * i + b
            in_copy(w, b)[1].wait()

            @pl.when(w >= 2)
            def _():
                out_copy(w - 2, b).wait()

            def grp_body(j, carry2):
                base = _D * 16 * j
                for d in range(_D):
                    v = inb[b][d, pl.ds(16 * j, 16)]
                    plsc.store_scatter(outb[b], [_D * lanes + (base + d)], v)
                return carry2

            lax.fori_loop(0, _W // 16, grp_body, 0)
            out_copy(w, b).start()

            @pl.when(w + 2 < _NWIN)
            def _():
                in_copy(w + 2, b)[1].start()
        return carry

    lax.fori_loop(0, _NWIN // 2, win_body, 0)
    for b in range(2):
        out_copy(_NWIN - 2 + b, b).wait()

    @pl.when(jnp.logical_and(cid == 1, sid == _NS - 1))
    def _():
        tin = pltpu.make_async_copy(tail_hbm, tail_v, si0)
        tin.start()
        tin.wait()

        def tail_grp(j, carry):
            base = _D * 16 * j
            for d in range(_D):
                v = tail_v[d, pl.ds(16 * j, 16)]
                plsc.store_scatter(ob0_v, [_D * lanes + (base + d)], v)
            return carry

        lax.fori_loop(0, 128 // 16, tail_grp, 0)
        tout = pltpu.make_async_copy(
            ob0_v.at[pl.ds(0, 128 * _D)],
            t_hbm.at[pl.ds((_VOCAB - 128) * _D, 128 * _D)], so0)
        tout.start()
        tout.wait()


_transpose = functools.partial(
    pl.kernel,
    mesh=plsc.VectorSubcoreMesh(core_axis_name="c", subcore_axis_name="s"),
    out_type=jax.ShapeDtypeStruct((_VOCAB * _D,), jnp.float32),
    scratch_types=[
        pltpu.VMEM((_D, _W), jnp.float32),
        pltpu.VMEM((_D, _W), jnp.float32),
        pltpu.VMEM((_W * _D,), jnp.float32),
        pltpu.VMEM((_W * _D,), jnp.float32),
        pltpu.VMEM((_D, 128), jnp.float32),
        pltpu.SemaphoreType.DMA,
        pltpu.SemaphoreType.DMA,
        pltpu.SemaphoreType.DMA,
        pltpu.SemaphoreType.DMA,
    ],
    compiler_params=pltpu.CompilerParams(use_tc_tiling_on_sc=True,
                                         needs_layout_passes=False),
)(_tr_body)


def _pool_body(x_hbm, emb_hbm, out_hbm, idx_v, rows0_v, rows1_v, stage_v,
               sem0, sem1):
    wid = lax.axis_index("s") * _NC + lax.axis_index("c")
    base = wid * _B_PER_W
    rows = (rows0_v, rows1_v)
    sems = (sem0, sem1)

    def gather(r, b):
        c1 = pltpu.make_async_copy(
            emb_hbm.at[idx_v.at[r, pl.ds(0, 128)]],
            rows[b].at[pl.ds(0, 128), :], sems[b])
        c2 = pltpu.make_async_copy(
            emb_hbm.at[idx_v.at[r, pl.ds(128, _L - 128)]],
            rows[b].at[pl.ds(128, _L - 128), :], sems[b])
        return c1, c2

    def reduce_store(r, b):
        def red_body(j, accs):
            a0, a1, a2, a3 = accs
            for k in range(_UNROLL):
                row = j * _UNROLL + k
                if k % 2 == 0:
                    a0 = a0 + rows[b][row, pl.ds(0, 16)]
                    a1 = a1 + rows[b][row, pl.ds(16, 16)]
                else:
                    a2 = a2 + rows[b][row, pl.ds(0, 16)]
                    a3 = a3 + rows[b][row, pl.ds(16, 16)]
            return a0, a1, a2, a3

        z = jnp.zeros((16,), jnp.float32)
        a0, a1, a2, a3 = lax.fori_loop(0, _L // _UNROLL, red_body,
                                       (z, z, z, z))
        stage_v[r, pl.ds(0, 16)] = a0 + a2
        stage_v[r, pl.ds(16, 16)] = a1 + a3

    for c in range(_B_PER_W // _CHUNK):
        cbase = base + c * _CHUNK
        pltpu.sync_copy(x_hbm.at[pl.ds(cbase, _CHUNK), :], idx_v)
        for b in range(2):
            c1, c2 = gather(b, b)
            c1.start()
            c2.start()

        def pair_body(i, carry):
            for b in range(2):
                r = 2 * i + b
                c1, c2 = gather(r, b)
                c1.wait()
                c2.wait()
                reduce_store(r, b)

                @pl.when(r + 2 < _CHUNK)
                def _():
                    n1, n2 = gather(r + 2, b)
                    n1.start()
                    n2.start()
            return carry

        lax.fori_loop(0, _CHUNK // 2, pair_body, 0)
        pltpu.sync_copy(stage_v, out_hbm.at[pl.ds(cbase, _CHUNK), :])


_pool = functools.partial(
    pl.kernel,
    mesh=plsc.VectorSubcoreMesh(core_axis_name="c", subcore_axis_name="s"),
    out_type=jax.ShapeDtypeStruct((_B, _D), jnp.float32),
    scratch_types=[
        pltpu.VMEM((_CHUNK, _L), jnp.int32),
        pltpu.VMEM((_L, _D), jnp.float32),
        pltpu.VMEM((_L, _D), jnp.float32),
        pltpu.VMEM((_CHUNK, _D), jnp.float32),
        pltpu.SemaphoreType.DMA,
        pltpu.SemaphoreType.DMA,
    ],
    compiler_params=pltpu.CompilerParams(use_tc_tiling_on_sc=False),
)(_pool_body)


def _mlp_body(s_ref, w1_ref, b1_ref, w2_ref, b2_ref, o_ref):
    s = s_ref[...] * (1.0 / _L)
    h = jnp.dot(s, w1_ref[...], preferred_element_type=jnp.float32)
    h = jnp.maximum(h + b1_ref[...], 0.0)
    o = jnp.dot(h, w2_ref[...], preferred_element_type=jnp.float32)
    o_ref[...] = jax.nn.sigmoid(o + b2_ref[...])


def _mlp(pooled, w1, b1, w2, b2):
    return pl.pallas_call(
        _mlp_body,
        out_shape=jax.ShapeDtypeStruct((_B, 1), jnp.float32),
    )(pooled, w1, b1.reshape(1, _H), w2, b2.reshape(1, 1))


def kernel(x, emb, W1, b1, W2, b2):
    tail_t = jnp.transpose(emb[_VOCAB - 128:, :])
    t1d = _transpose(emb.T, tail_t)
    pooled = _pool(x, t1d.reshape(_VOCAB, _D))
    return _mlp(pooled, W1, b1, W2, b2)


# speedup vs baseline: 3.9753x; 1.1719x over previous
import functools

import jax
import jax.numpy as jnp
from jax import lax
from jax.experimental import pallas as pl
from jax.experimental.pallas import tpu as pltpu
from jax.experimental.pallas import tpu_sc as plsc

_VOCAB = 1000000
_D = 32
_H = 64
_B = 16384
_L = 200

_NC = 2
_NS = 16
_NW = _NC * _NS
_B_PER_W = _B // _NW
_CHUNK = 256
_UNROLL = 25

_W = 512
_SPLIT = 499968
_EDGE = (_VOCAB // 128) * 128
_NWIN = 62


def _tr_body(embt_hbm, tail_hbm, t_hbm, in0_v, in1_v, ob0_v, ob1_v, tail_v,
             si0, si1, so0, so1):
    cid = lax.axis_index("c")
    sid = lax.axis_index("s")
    lanes = lax.iota(jnp.int32, 16)
    inb = (in0_v, in1_v)
    outb = (ob0_v, ob1_v)
    sin = (si0, si1)
    sout = (so0, so1)

    lo = jnp.where(cid == 0, 0, _SPLIT)
    hi = jnp.where(cid == 0, _SPLIT, _EDGE)
    span = hi - lo
    cbase = lo + ((sid * span) // (_NS * 128)) * 128
    top = hi - _W

    def in_copy(w, b):
        col0 = jnp.minimum(cbase + _W * w, top)
        return col0, pltpu.make_async_copy(
            embt_hbm.at[:, pl.ds(col0, _W)], inb[b], sin[b])

    def out_copy(w, b):
        col0 = jnp.minimum(cbase + _W * w, top)
        return pltpu.make_async_copy(
            outb[b], t_hbm.at[pl.ds(col0 * _D, _W * _D)], sout[b])

    for b in range(2):
        in_copy(b, b)[1].start()

    def win_body(i, carry):
        for b in range(2):
            w = 2 * i + b
            in_copy(w, b)[1].wait()

            @pl.when(w >= 2)
            def _():
                out_copy(w - 2, b).wait()

            def grp_body(j, carry2):
                base = _D * 16 * j
                for d0 in range(0, _D, 8):
                    vs = [inb[b][d0 + q, pl.ds(16 * j, 16)]
                          for q in range(8)]
                    for q in range(8):
                        plsc.store_scatter(
                            outb[b], [_D * lanes + (base + d0 + q)], vs[q])
                return carry2

            lax.fori_loop(0, _W // 16, grp_body, 0)
            out_copy(w, b).start()

            @pl.when(w + 2 < _NWIN)
            def _():
                in_copy(w + 2, b)[1].start()
        return carry

    lax.fori_loop(0, _NWIN // 2, win_body, 0)
    for b in range(2):
        out_copy(_NWIN - 2 + b, b).wait()

    @pl.when(jnp.logical_and(cid == 1, sid == _NS - 1))
    def _():
        tin = pltpu.make_async_copy(tail_hbm, tail_v, si0)
        tin.start()
        tin.wait()

        def tail_grp(j, carry):
            base = _D * 16 * j
            for d0 in range(0, _D, 8):
                vs = [tail_v[d0 + q, pl.ds(16 * j, 16)] for q in range(8)]
                for q in range(8):
                    plsc.store_scatter(
                        ob0_v, [_D * lanes + (base + d0 + q)], vs[q])
            return carry

        lax.fori_loop(0, 128 // 16, tail_grp, 0)
        tout = pltpu.make_async_copy(
            ob0_v.at[pl.ds(0, 128 * _D)],
            t_hbm.at[pl.ds((_VOCAB - 128) * _D, 128 * _D)], so0)
        tout.start()
        tout.wait()


_transpose = functools.partial(
    pl.kernel,
    mesh=plsc.VectorSubcoreMesh(core_axis_name="c", subcore_axis_name="s"),
    out_type=jax.ShapeDtypeStruct((_VOCAB * _D,), jnp.float32),
    scratch_types=[
        pltpu.VMEM((_D, _W), jnp.float32),
        pltpu.VMEM((_D, _W), jnp.float32),
        pltpu.VMEM((_W * _D,), jnp.float32),
        pltpu.VMEM((_W * _D,), jnp.float32),
        pltpu.VMEM((_D, 128), jnp.float32),
        pltpu.SemaphoreType.DMA,
        pltpu.SemaphoreType.DMA,
        pltpu.SemaphoreType.DMA,
        pltpu.SemaphoreType.DMA,
    ],
    compiler_params=pltpu.CompilerParams(use_tc_tiling_on_sc=True,
                                         needs_layout_passes=False),
)(_tr_body)


def _pool_body(x_hbm, emb_hbm, out_hbm, idx_v, rows0_v, rows1_v, stage_v,
               sem0, sem1):
    wid = lax.axis_index("s") * _NC + lax.axis_index("c")
    base = wid * _B_PER_W
    rows = (rows0_v, rows1_v)
    sems = (sem0, sem1)

    def gather(r, b):
        c1 = pltpu.make_async_copy(
            emb_hbm.at[idx_v.at[r, pl.ds(0, 128)]],
            rows[b].at[pl.ds(0, 128), :], sems[b])
        c2 = pltpu.make_async_copy(
            emb_hbm.at[idx_v.at[r, pl.ds(128, _L - 128)]],
            rows[b].at[pl.ds(128, _L - 128), :], sems[b])
        return c1, c2

    def reduce_store(r, b):
        def red_body(j, accs):
            a0, a1, a2, a3 = accs
            for k in range(_UNROLL):
                row = j * _UNROLL + k
                if k % 2 == 0:
                    a0 = a0 + rows[b][row, pl.ds(0, 16)]
                    a1 = a1 + rows[b][row, pl.ds(16, 16)]
                else:
                    a2 = a2 + rows[b][row, pl.ds(0, 16)]
                    a3 = a3 + rows[b][row, pl.ds(16, 16)]
            return a0, a1, a2, a3

        z = jnp.zeros((16,), jnp.float32)
        a0, a1, a2, a3 = lax.fori_loop(0, _L // _UNROLL, red_body,
                                       (z, z, z, z))
        stage_v[r, pl.ds(0, 16)] = a0 + a2
        stage_v[r, pl.ds(16, 16)] = a1 + a3

    for c in range(_B_PER_W // _CHUNK):
        cbase = base + c * _CHUNK
        pltpu.sync_copy(x_hbm.at[pl.ds(cbase, _CHUNK), :], idx_v)
        for b in range(2):
            c1, c2 = gather(b, b)
            c1.start()
            c2.start()

        def pair_body(i, carry):
            for b in range(2):
                r = 2 * i + b
                c1, c2 = gather(r, b)
                c1.wait()
                c2.wait()
                reduce_store(r, b)

                @pl.when(r + 2 < _CHUNK)
                def _():
                    n1, n2 = gather(r + 2, b)
                    n1.start()
                    n2.start()
            return carry

        lax.fori_loop(0, _CHUNK // 2, pair_body, 0)
        pltpu.sync_copy(stage_v, out_hbm.at[pl.ds(cbase, _CHUNK), :])


_pool = functools.partial(
    pl.kernel,
    mesh=plsc.VectorSubcoreMesh(core_axis_name="c", subcore_axis_name="s"),
    out_type=jax.ShapeDtypeStruct((_B, _D), jnp.float32),
    scratch_types=[
        pltpu.VMEM((_CHUNK, _L), jnp.int32),
        pltpu.VMEM((_L, _D), jnp.float32),
        pltpu.VMEM((_L, _D), jnp.float32),
        pltpu.VMEM((_CHUNK, _D), jnp.float32),
        pltpu.SemaphoreType.DMA,
        pltpu.SemaphoreType.DMA,
    ],
    compiler_params=pltpu.CompilerParams(use_tc_tiling_on_sc=False),
)(_pool_body)


def _mlp_body(s_ref, w1_ref, b1_ref, w2_ref, b2_ref, o_ref):
    s = s_ref[...] * (1.0 / _L)
    h = jnp.dot(s, w1_ref[...], preferred_element_type=jnp.float32)
    h = jnp.maximum(h + b1_ref[...], 0.0)
    o = jnp.dot(h, w2_ref[...], preferred_element_type=jnp.float32)
    o_ref[...] = jax.nn.sigmoid(o + b2_ref[...])


def _mlp(pooled, w1, b1, w2, b2):
    return pl.pallas_call(
        _mlp_body,
        out_shape=jax.ShapeDtypeStruct((_B, 1), jnp.float32),
    )(pooled, w1, b1.reshape(1, _H), w2, b2.reshape(1, 1))


def kernel(x, emb, W1, b1, W2, b2):
    tail_t = jnp.transpose(emb[_VOCAB - 128:, :])
    t1d = _transpose(emb.T, tail_t)
    pooled = _pool(x, t1d.reshape(_VOCAB, _D))
    return _mlp(pooled, W1, b1, W2, b2)


# speedup vs baseline: 5.6316x; 1.4166x over previous
import functools

import jax
import jax.numpy as jnp
from jax import lax
from jax.experimental import pallas as pl
from jax.experimental.pallas import tpu as pltpu
from jax.experimental.pallas import tpu_sc as plsc

_VOCAB = 1000000
_D = 32
_H = 64
_B = 16384
_L = 200

_NC = 2
_NS = 16
_NW = _NC * _NS
_B_PER_W = _B // _NW
_CHUNK = 256
_UNROLL = 25

_W = 512
_DP = 33
_SPLIT = 499968
_EDGE = (_VOCAB // 128) * 128
_NWIN = 62


def _tr_body(embt_hbm, tail_hbm, t_hbm, in0_v, in1_v, ob0_v, ob1_v, pad_v,
             tail_v, si0, si1, so0, so1):
    cid = lax.axis_index("c")
    sid = lax.axis_index("s")
    lanes = lax.iota(jnp.int32, 16)
    inb = (in0_v, in1_v)
    outb = (ob0_v, ob1_v)
    sin = (si0, si1)
    sout = (so0, so1)

    lo = jnp.where(cid == 0, 0, _SPLIT)
    hi = jnp.where(cid == 0, _SPLIT, _EDGE)
    span = hi - lo
    cbase = lo + ((sid * span) // (_NS * 128)) * 128
    top = hi - _W

    def in_copy(w, b):
        col0 = jnp.minimum(cbase + _W * w, top)
        return col0, pltpu.make_async_copy(
            embt_hbm.at[:, pl.ds(col0, _W)], inb[b], sin[b])

    def out_copy(w, b):
        col0 = jnp.minimum(cbase + _W * w, top)
        return pltpu.make_async_copy(
            outb[b], t_hbm.at[pl.ds(col0 * _D, _W * _D)], sout[b])

    for b in range(2):
        in_copy(b, b)[1].start()

    def win_body(i, carry):
        for b in range(2):
            w = 2 * i + b
            in_copy(w, b)[1].wait()

            @pl.when(w >= 2)
            def _():
                out_copy(w - 2, b).wait()

            def grp_body(j, carry2):
                base = _DP * 16 * j
                for d0 in range(0, _D, 8):
                    vs = [inb[b][d0 + q, pl.ds(16 * j, 16)]
                          for q in range(8)]
                    for q in range(8):
                        plsc.store_scatter(
                            pad_v, [_DP * lanes + (base + d0 + q)], vs[q])
                return carry2

            lax.fori_loop(0, _W // 16, grp_body, 0)

            def col_body(j, carry2):
                for q in range(4):
                    col = 4 * j + q
                    g0 = plsc.load_gather(pad_v, [_DP * col + lanes])
                    g1 = plsc.load_gather(pad_v, [_DP * col + 16 + lanes])
                    outb[b][pl.ds(_D * col, 16)] = g0
                    outb[b][pl.ds(_D * col + 16, 16)] = g1
                return carry2

            lax.fori_loop(0, _W // 4, col_body, 0)
            out_copy(w, b).start()

            @pl.when(w + 2 < _NWIN)
            def _():
                in_copy(w + 2, b)[1].start()
        return carry

    lax.fori_loop(0, _NWIN // 2, win_body, 0)
    for b in range(2):
        out_copy(_NWIN - 2 + b, b).wait()

    @pl.when(jnp.logical_and(cid == 1, sid == _NS - 1))
    def _():
        tin = pltpu.make_async_copy(tail_hbm, tail_v, si0)
        tin.start()
        tin.wait()

        def tail_grp(j, carry):
            base = _D * 16 * j
            for d0 in range(0, _D, 8):
                vs = [tail_v[d0 + q, pl.ds(16 * j, 16)] for q in range(8)]
                for q in range(8):
                    plsc.store_scatter(
                        ob0_v, [_D * lanes + (base + d0 + q)], vs[q])
            return carry

        lax.fori_loop(0, 128 // 16, tail_grp, 0)
        tout = pltpu.make_async_copy(
            ob0_v.at[pl.ds(0, 128 * _D)],
            t_hbm.at[pl.ds((_VOCAB - 128) * _D, 128 * _D)], so0)
        tout.start()
        tout.wait()


_transpose = functools.partial(
    pl.kernel,
    mesh=plsc.VectorSubcoreMesh(core_axis_name="c", subcore_axis_name="s"),
    out_type=jax.ShapeDtypeStruct((_VOCAB * _D,), jnp.float32),
    scratch_types=[
        pltpu.VMEM((_D, _W), jnp.float32),
        pltpu.VMEM((_D, _W), jnp.float32),
        pltpu.VMEM((_W * _D,), jnp.float32),
        pltpu.VMEM((_W * _D,), jnp.float32),
        pltpu.VMEM((_W * _DP,), jnp.float32),
        pltpu.VMEM((_D, 128), jnp.float32),
        pltpu.SemaphoreType.DMA,
        pltpu.SemaphoreType.DMA,
        pltpu.SemaphoreType.DMA,
        pltpu.SemaphoreType.DMA,
    ],
    compiler_params=pltpu.CompilerParams(use_tc_tiling_on_sc=True,
                                         needs_layout_passes=False),
)(_tr_body)


def _pool_body(x_hbm, emb_hbm, out_hbm, idx_v, rows0_v, rows1_v, stage_v,
               sem0, sem1):
    wid = lax.axis_index("s") * _NC + lax.axis_index("c")
    base = wid * _B_PER_W
    rows = (rows0_v, rows1_v)
    sems = (sem0, sem1)

    def gather(r, b):
        c1 = pltpu.make_async_copy(
            emb_hbm.at[idx_v.at[r, pl.ds(0, 128)]],
            rows[b].at[pl.ds(0, 128), :], sems[b])
        c2 = pltpu.make_async_copy(
            emb_hbm.at[idx_v.at[r, pl.ds(128, _L - 128)]],
            rows[b].at[pl.ds(128, _L - 128), :], sems[b])
        return c1, c2

    def reduce_store(r, b):
        def red_body(j, accs):
            a0, a1, a2, a3 = accs
            for k in range(_UNROLL):
                row = j * _UNROLL + k
                if k % 2 == 0:
                    a0 = a0 + rows[b][row, pl.ds(0, 16)]
                    a1 = a1 + rows[b][row, pl.ds(16, 16)]
                else:
                    a2 = a2 + rows[b][row, pl.ds(0, 16)]
                    a3 = a3 + rows[b][row, pl.ds(16, 16)]
            return a0, a1, a2, a3

        z = jnp.zeros((16,), jnp.float32)
        a0, a1, a2, a3 = lax.fori_loop(0, _L // _UNROLL, red_body,
                                       (z, z, z, z))
        stage_v[r, pl.ds(0, 16)] = a0 + a2
        stage_v[r, pl.ds(16, 16)] = a1 + a3

    for c in range(_B_PER_W // _CHUNK):
        cbase = base + c * _CHUNK
        pltpu.sync_copy(x_hbm.at[pl.ds(cbase, _CHUNK), :], idx_v)
        for b in range(2):
            c1, c2 = gather(b, b)
            c1.start()
            c2.start()

        def pair_body(i, carry):
            for b in range(2):
                r = 2 * i + b
                c1, c2 = gather(r, b)
                c1.wait()
                c2.wait()
                reduce_store(r, b)

                @pl.when(r + 2 < _CHUNK)
                def _():
                    n1, n2 = gather(r + 2, b)
                    n1.start()
                    n2.start()
            return carry

        lax.fori_loop(0, _CHUNK // 2, pair_body, 0)
        pltpu.sync_copy(stage_v, out_hbm.at[pl.ds(cbase, _CHUNK), :])


_pool = functools.partial(
    pl.kernel,
    mesh=plsc.VectorSubcoreMesh(core_axis_name="c", subcore_axis_name="s"),
    out_type=jax.ShapeDtypeStruct((_B, _D), jnp.float32),
    scratch_types=[
        pltpu.VMEM((_CHUNK, _L), jnp.int32),
        pltpu.VMEM((_L, _D), jnp.float32),
        pltpu.VMEM((_L, _D), jnp.float32),
        pltpu.VMEM((_CHUNK, _D), jnp.float32),
        pltpu.SemaphoreType.DMA,
        pltpu.SemaphoreType.DMA,
    ],
    compiler_params=pltpu.CompilerParams(use_tc_tiling_on_sc=False),
)(_pool_body)


def _mlp_body(s_ref, w1_ref, b1_ref, w2_ref, b2_ref, o_ref):
    s = s_ref[...] * (1.0 / _L)
    h = jnp.dot(s, w1_ref[...], preferred_element_type=jnp.float32)
    h = jnp.maximum(h + b1_ref[...], 0.0)
    o = jnp.dot(h, w2_ref[...], preferred_element_type=jnp.float32)
    o_ref[...] = jax.nn.sigmoid(o + b2_ref[...])


def _mlp(pooled, w1, b1, w2, b2):
    return pl.pallas_call(
        _mlp_body,
        out_shape=jax.ShapeDtypeStruct((_B, 1), jnp.float32),
    )(pooled, w1, b1.reshape(1, _H), w2, b2.reshape(1, 1))


def kernel(x, emb, W1, b1, W2, b2):
    tail_t = jnp.transpose(emb[_VOCAB - 128:, :])
    t1d = _transpose(emb.T, tail_t)
    pooled = _pool(x, t1d.reshape(_VOCAB, _D))
    return _mlp(pooled, W1, b1, W2, b2)
